# Initial kernel scaffold; baseline (speedup 1.0000x reference)
import jax
import jax.numpy as jnp
from jax.experimental import pallas as pl


def kernel(features, index_labels, proxy_memory, img_proxy_index, all_proxy_label):
    raise NotImplementedError("write your pallas kernel here")


# speedup vs baseline: 4.1114x; 4.1114x over previous
import jax
import jax.numpy as jnp
from jax import lax
from jax.experimental import pallas as pl
from jax.experimental.pallas import tpu as pltpu

NEGK = 50
TEMP = 0.05
NEG_FILL = -3.0e38


def _loss_block(feat_ref, pm_ref, labels_ref, blab_ref, out_ref, scores_ref):
    rb, d = feat_ref.shape
    m = pm_ref.shape[0]
    cb = min(512, m)
    n_chunks = m // cb

    f = feat_ref[...]
    blab = blab_ref[...]

    def chunk(c, carry):
        n_pos, sum_pos, pos_max, pos_se = carry
        pm_c = pm_ref[pl.ds(c * cb, cb), :]
        s = lax.dot_general(
            f, pm_c, (((1,), (1,)), ((), ())),
            preferred_element_type=jnp.float32) * (1.0 / TEMP)
        lab_c = labels_ref[:, pl.ds(c * cb, cb)]
        mask = lab_c == blab
        scores_ref[:, pl.ds(c * cb, cb)] = jnp.where(mask, NEG_FILL, s)
        n_pos = n_pos + jnp.sum(mask.astype(jnp.float32), axis=1, keepdims=True)
        sum_pos = sum_pos + jnp.sum(jnp.where(mask, s, 0.0), axis=1, keepdims=True)
        m_new = jnp.maximum(
            pos_max, jnp.max(jnp.where(mask, s, NEG_FILL), axis=1, keepdims=True))
        pos_se = pos_se * jnp.exp(pos_max - m_new) + jnp.sum(
            jnp.where(mask, jnp.exp(s - m_new), 0.0), axis=1, keepdims=True)
        return n_pos, sum_pos, m_new, pos_se

    zero = jnp.zeros((rb, 1), jnp.float32)
    n_pos, sum_pos, pos_max, pos_se = lax.fori_loop(
        0, n_chunks, chunk, (zero, zero, jnp.full((rb, 1), NEG_FILL), zero))

    x0 = scores_ref[...]
    neg_max = jnp.max(x0, axis=1, keepdims=True)
    z = jnp.maximum(neg_max, pos_max)
    kneg = jnp.maximum(jnp.float32(NEGK) - jnp.minimum(n_pos, jnp.float32(NEGK)), 0.0)

    def extract(_, carry):
        t, cum, se_neg = carry
        x = scores_ref[...]
        mx = jnp.max(jnp.where(x < t, x, NEG_FILL), axis=1, keepdims=True)
        cnt = jnp.sum(jnp.where(x == mx, 1.0, 0.0), axis=1, keepdims=True)
        w = jnp.clip(kneg - cum, 0.0, cnt)
        se_neg = se_neg + w * jnp.exp(mx - z)
        return mx, cum + cnt, se_neg

    big = jnp.full((rb, 1), jnp.float32(3.0e38))
    _, _, se_neg = lax.fori_loop(0, NEGK, extract, (big, zero, zero))

    se_total = pos_se * jnp.exp(pos_max - z) + se_neg
    lse = z + jnp.log(se_total)
    k_pos = jnp.minimum(n_pos, jnp.float32(NEGK))
    loss_rows = (k_pos / n_pos) * lse - sum_pos / n_pos
    out_ref[...] = jnp.sum(loss_rows).reshape(1, 1, 1)


def _build_loss_call(b, m, d, rb, interpret=False):
    grid = (b // rb,)
    return pl.pallas_call(
        _loss_block,
        grid=grid,
        in_specs=[
            pl.BlockSpec((rb, d), lambda r: (r, 0)),
            pl.BlockSpec((m, d), lambda r: (0, 0)),
            pl.BlockSpec((1, m), lambda r: (0, 0)),
            pl.BlockSpec((rb, 1), lambda r: (r, 0)),
        ],
        out_specs=pl.BlockSpec((1, 1, 1), lambda r: (r, 0, 0)),
        out_shape=jax.ShapeDtypeStruct((b // rb, 1, 1), jnp.float32),
        scratch_shapes=[pltpu.VMEM((rb, m), jnp.float32)],
        interpret=interpret,
    )


def kernel(features, index_labels, proxy_memory, img_proxy_index, all_proxy_label):
    b, d = features.shape
    m = proxy_memory.shape[0]
    rb = min(128, b)
    batch_pseudo_label = jnp.take(
        all_proxy_label, jnp.take(img_proxy_index, index_labels))
    call = _build_loss_call(b, m, d, rb)
    partial = call(
        features, proxy_memory, all_proxy_label.reshape(1, m),
        batch_pseudo_label.reshape(b, 1))
    return jnp.sum(partial) / b


# speedup vs baseline: 7.9983x; 1.9454x over previous
import jax
import jax.numpy as jnp
from jax import lax
from jax.experimental import pallas as pl
from jax.experimental.pallas import tpu as pltpu

NEGK = 50
TEMP = 0.05
NEG_FILL = -3.0e38
IMIN = -2147483648
IMAX = 2147483647


def _to_key(s):
    bits = lax.bitcast_convert_type(s, jnp.int32)
    return jnp.where(bits < 0, bits ^ IMAX, bits)


def _from_key(k):
    bits = jnp.where(k < 0, k ^ IMAX, k)
    return lax.bitcast_convert_type(bits, jnp.float32)


def _loss_block(feat_ref, pm_ref, labels_ref, blab_ref, out_ref, keys_ref):
    rb, d = feat_ref.shape
    m = pm_ref.shape[0]
    cb = min(512, m)
    n_chunks = m // cb

    f = feat_ref[...]
    blab = blab_ref[...]

    def chunk(c, carry):
        n_pos, sum_pos, pos_max, pos_se = carry
        pm_c = pm_ref[pl.ds(c * cb, cb), :]
        s = lax.dot_general(
            f, pm_c, (((1,), (1,)), ((), ())),
            preferred_element_type=jnp.float32) * (1.0 / TEMP)
        mask = labels_ref[:, pl.ds(c * cb, cb)] == blab
        keys_ref[:, pl.ds(c * cb, cb)] = jnp.where(mask, IMIN, _to_key(s))
        n_pos = n_pos + jnp.sum(mask.astype(jnp.int32), axis=1, keepdims=True)
        sum_pos = sum_pos + jnp.sum(jnp.where(mask, s, 0.0), axis=1, keepdims=True)
        m_new = jnp.maximum(
            pos_max, jnp.max(jnp.where(mask, s, NEG_FILL), axis=1, keepdims=True))
        pos_se = pos_se * jnp.exp(pos_max - m_new) + jnp.sum(
            jnp.where(mask, jnp.exp(s - m_new), 0.0), axis=1, keepdims=True)
        return n_pos, sum_pos, m_new, pos_se

    zero = jnp.zeros((rb, 1), jnp.float32)
    n_pos, sum_pos, pos_max, pos_se = lax.fori_loop(
        0, n_chunks, chunk,
        (jnp.zeros((rb, 1), jnp.int32), zero, jnp.full((rb, 1), NEG_FILL), zero))

    keys0 = keys_ref[...]
    hi0 = jnp.max(keys0, axis=1, keepdims=True)
    lo0 = jnp.min(jnp.where(keys0 == IMIN, IMAX, keys0), axis=1, keepdims=True)
    kneg = jnp.maximum(NEGK - jnp.minimum(n_pos, NEGK), 0)

    def search(_, carry):
        lo, hi = carry
        mid = (lo >> 1) + (hi >> 1) + (lo & hi & 1) + ((lo ^ hi) & 1)
        cnt = jnp.sum((keys_ref[...] >= mid).astype(jnp.int32),
                      axis=1, keepdims=True)
        ge = cnt >= kneg
        return jnp.where(ge, mid, lo), jnp.where(ge, hi, mid - 1)

    t_key, _ = lax.fori_loop(0, 32, search, (lo0, hi0))

    z = jnp.maximum(_from_key(hi0), pos_max)
    keys = keys_ref[...]
    gt = keys > t_key
    cnt_gt = jnp.sum(gt.astype(jnp.float32), axis=1, keepdims=True)
    se_gt = jnp.sum(jnp.where(gt, jnp.exp(_from_key(keys) - z), 0.0),
                    axis=1, keepdims=True)
    kneg_f = kneg.astype(jnp.float32)
    se_neg = se_gt + (kneg_f - cnt_gt) * jnp.exp(_from_key(t_key) - z)

    se_total = pos_se * jnp.exp(pos_max - z) + se_neg
    lse = z + jnp.log(se_total)
    n_pos_f = n_pos.astype(jnp.float32)
    k_pos = jnp.minimum(n_pos_f, jnp.float32(NEGK))
    loss_rows = (k_pos / n_pos_f) * lse - sum_pos / n_pos_f
    out_ref[...] = jnp.sum(loss_rows).reshape(1, 1, 1)


def _build_loss_call(b, m, d, rb, interpret=False):
    grid = (b // rb,)
    return pl.pallas_call(
        _loss_block,
        grid=grid,
        in_specs=[
            pl.BlockSpec((rb, d), lambda r: (r, 0)),
            pl.BlockSpec((m, d), lambda r: (0, 0)),
            pl.BlockSpec((1, m), lambda r: (0, 0)),
            pl.BlockSpec((rb, 1), lambda r: (r, 0)),
        ],
        out_specs=pl.BlockSpec((1, 1, 1), lambda r: (r, 0, 0)),
        out_shape=jax.ShapeDtypeStruct((b // rb, 1, 1), jnp.float32),
        scratch_shapes=[pltpu.VMEM((rb, m), jnp.int32)],
        interpret=interpret,
    )


def kernel(features, index_labels, proxy_memory, img_proxy_index, all_proxy_label):
    b, d = features.shape
    m = proxy_memory.shape[0]
    rb = min(128, b)
    batch_pseudo_label = jnp.take(
        all_proxy_label, jnp.take(img_proxy_index, index_labels))
    call = _build_loss_call(b, m, d, rb)
    partial = call(
        features, proxy_memory, all_proxy_label.reshape(1, m),
        batch_pseudo_label.reshape(b, 1))
    return jnp.sum(partial) / b
